# baseline (speedup 1.0000x reference)
import functools

import jax
import jax.numpy as jnp
from jax.experimental import pallas as pl
from jax.experimental.pallas import tpu as pltpu

_B = 16384
_D = 64
_W = 128
_R = _B * _D // _W
_BLK = 1024
_GRID = _R // _BLK


def _body(gu_ref, gi_ref, m2_ref, xui_ref, gamu_ref, gami_ref):
    gu = gu_ref[...]
    gi = gi_ref[...]
    gamu_ref[...] = gu
    gami_ref[...] = gi
    xui_ref[...] = jax.lax.dot_general(
        gu * gi, m2_ref[...], (((1,), (0,)), ((), ())),
        preferred_element_type=jnp.float32)


@jax.jit
def _uuii_tc(gu2, gi2):
    m2 = jnp.repeat(jnp.eye(2, dtype=jnp.float32), _D, axis=0)
    return pl.pallas_call(
        _body,
        grid=(_GRID,),
        in_specs=[
            pl.BlockSpec((_BLK, _W), lambda i: (i, 0)),
            pl.BlockSpec((_BLK, _W), lambda i: (i, 0)),
            pl.BlockSpec((_W, 2), lambda i: (0, 0)),
        ],
        out_specs=[
            pl.BlockSpec((_BLK, 2), lambda i: (i, 0)),
            pl.BlockSpec((_BLK, _W), lambda i: (i, 0)),
            pl.BlockSpec((_BLK, _W), lambda i: (i, 0)),
        ],
        out_shape=[
            jax.ShapeDtypeStruct((_R, 2), jnp.float32),
            jax.ShapeDtypeStruct((_R, _W), jnp.float32),
            jax.ShapeDtypeStruct((_R, _W), jnp.float32),
        ],
        compiler_params=pltpu.CompilerParams(
            dimension_semantics=("arbitrary",),
        ),
    )(gu2, gi2, m2)


def kernel(gu, gi):
    xui2, gamu2, gami2 = _uuii_tc(gu.reshape(_R, _W), gi.reshape(_R, _W))
    return (xui2.reshape(_B), gamu2.reshape(_B, _D), gami2.reshape(_B, _D))
